# baseline (speedup 1.0000x reference)
import jax
import jax.numpy as jnp
from jax.experimental import pallas as pl

_BT = 512


def _router_block(xa_ref, xb_ref, w_ref, b_ref, o_ref):
    wb = w_ref[...].astype(jnp.bfloat16)
    bias = jnp.transpose(b_ref[...], (1, 0))
    xa = xa_ref[...].astype(jnp.bfloat16)
    acc_a = jax.lax.dot_general(
        wb, xa, (((1,), (1,)), ((), ())),
        preferred_element_type=jnp.float32)
    o_ref[:, :_BT] = acc_a + bias
    xb = xb_ref[...].astype(jnp.bfloat16)
    acc_b = jax.lax.dot_general(
        wb, xb, (((1,), (1,)), ((), ())),
        preferred_element_type=jnp.float32)
    o_ref[:, _BT:] = acc_b + bias


def kernel(x, W, b):
    tokens, d = x.shape
    e = W.shape[0]
    b2 = b.reshape(1, e)
    logits_t = pl.pallas_call(
        _router_block,
        grid=(tokens // (2 * _BT),),
        in_specs=[
            pl.BlockSpec((_BT, d), lambda i: (2 * i, 0)),
            pl.BlockSpec((_BT, d), lambda i: (2 * i + 1, 0)),
            pl.BlockSpec((e, d), lambda i: (0, 0)),
            pl.BlockSpec((1, e), lambda i: (0, 0)),
        ],
        out_specs=pl.BlockSpec((e, 2 * _BT), lambda i: (0, i)),
        out_shape=jax.ShapeDtypeStruct((e, tokens), jnp.float32),
    )(x, x, W, b2)
    return logits_t.T
